# baseline (speedup 1.0000x reference)
import functools

import jax
import jax.numpy as jnp
from jax import lax
from jax.experimental import pallas as pl
from jax.experimental.pallas import tpu as pltpu
from jax.experimental.pallas import tpu_sc as plsc

B, L = 1024, 256
NC, NS = 2, 16
NW = NC * NS
RPW = B // NW
LANES = 16
GROUPS = RPW // LANES
UNROLL = 16

_mesh = plsc.VectorSubcoreMesh(core_axis_name="c", subcore_axis_name="s")


@functools.partial(
    pl.kernel,
    mesh=_mesh,
    compiler_params=pltpu.CompilerParams(needs_layout_passes=False),
    out_type=jax.ShapeDtypeStruct((NW, LANES), jnp.float32),
    scratch_types=[
        pltpu.VMEM((RPW * L,), jnp.float32),
        pltpu.VMEM((RPW * L,), jnp.int32),
        pltpu.VMEM((LANES,), jnp.float32),
        pltpu.SemaphoreType.DMA,
        pltpu.SemaphoreType.DMA,
    ],
)
def _bpmll_partials(x_hbm, t_hbm, out_hbm, x_v, t_v, o_v, sem_x, sem_t):
    wid = lax.axis_index("s") * NC + lax.axis_index("c")
    base = wid * (RPW * L)
    cx = pltpu.async_copy(x_hbm.at[pl.ds(base, RPW * L)], x_v, sem_x)
    ct = pltpu.async_copy(t_hbm.at[pl.ds(base, RPW * L)], t_v, sem_t)
    cx.wait()
    ct.wait()

    lanes = lax.iota(jnp.int32, LANES)
    zf = jnp.zeros((LANES,), jnp.float32)
    zi = jnp.zeros((LANES,), jnp.int32)
    acc = zf
    for g in range(GROUPS):
        rowoff = (lanes + g * LANES) * L

        def col_block(ci, carry):
            tot, spos, cnt, idxv = carry
            for j in range(UNROLL):
                idx = idxv if j == 0 else idxv + j
                xv = plsc.load_gather(x_v, [idx])
                tv = plsc.load_gather(t_v, [idx])
                fx = plsc.bitcast(plsc.bitcast(xv, jnp.int32) ^ (tv << 31),
                                  jnp.float32)
                e = jnp.exp(fx)
                tot = tot + e
                spos = spos + jnp.where(tv == 1, e, zf)
                cnt = cnt + tv
            return tot, spos, cnt, idxv + UNROLL

        tot, spos, cnt, _ = lax.fori_loop(
            0, L // UNROLL, col_block, (zf, zf, zi, rowoff))
        cntf = cnt.astype(jnp.float32)
        sneg = tot - spos
        acc = acc + (sneg * spos) / (cntf * (jnp.float32(L) - cntf))
    o_v[...] = acc * jnp.float32(1.0 / B)
    pltpu.sync_copy(o_v, out_hbm.at[wid])


def kernel(input, target):
    parts = _bpmll_partials(input.reshape(-1),
                            target.astype(jnp.int32).reshape(-1))
    return parts.sum()


# speedup vs baseline: 1.3386x; 1.3386x over previous
import functools

import jax
import jax.numpy as jnp
from jax import lax
from jax.experimental import pallas as pl
from jax.experimental.pallas import tpu as pltpu
from jax.experimental.pallas import tpu_sc as plsc

B, L = 1024, 256
NC, NS = 2, 16
NW = NC * NS
RPW = B // NW
LANES = 16
NCH = L // LANES

_mesh = plsc.VectorSubcoreMesh(core_axis_name="c", subcore_axis_name="s")


@functools.partial(
    pl.kernel,
    mesh=_mesh,
    compiler_params=pltpu.CompilerParams(needs_layout_passes=False),
    out_type=jax.ShapeDtypeStruct((NW, LANES), jnp.float32),
    scratch_types=[
        pltpu.VMEM((RPW, L), jnp.float32),
        pltpu.VMEM((RPW, L), jnp.int32),
        pltpu.VMEM((LANES,), jnp.float32),
        pltpu.SemaphoreType.DMA,
        pltpu.SemaphoreType.DMA,
    ],
)
def _bpmll_partials(x_hbm, t_hbm, out_hbm, x_v, t_v, o_v, sem_x, sem_t):
    wid = lax.axis_index("s") * NC + lax.axis_index("c")
    base = wid * RPW
    cx = pltpu.async_copy(x_hbm.at[pl.ds(base, RPW)], x_v, sem_x)
    ct = pltpu.async_copy(t_hbm.at[pl.ds(base, RPW)], t_v, sem_t)
    cx.wait()
    ct.wait()

    zf = jnp.zeros((LANES,), jnp.float32)
    zi = jnp.zeros((LANES,), jnp.int32)

    def row_body(r, acc):
        tot = zf
        spos = zf
        cnt = zi
        for j in range(NCH):
            xv = x_v[r, pl.ds(j * LANES, LANES)]
            tv = t_v[r, pl.ds(j * LANES, LANES)]
            fx = plsc.bitcast(plsc.bitcast(xv, jnp.int32) ^ (tv << 31),
                              jnp.float32)
            e = jnp.exp(fx)
            tot = tot + e
            spos = spos + jnp.where(tv == 1, e, zf)
            cnt = cnt + tv
        sp = jnp.sum(spos)
        sneg = jnp.sum(tot) - sp
        npos = jnp.sum(cnt.astype(jnp.float32))
        num = sneg * sp
        den = npos * (jnp.float32(L) - npos)
        numv = jnp.full((LANES,), num, jnp.float32)
        denv = jnp.full((LANES,), den, jnp.float32)
        return acc + numv / denv

    acc = lax.fori_loop(0, RPW, row_body, zf)
    lane = lax.iota(jnp.int32, LANES)
    o_v[...] = jnp.where(lane == 0, acc * jnp.float32(1.0 / B), 0.0)
    pltpu.sync_copy(o_v, out_hbm.at[wid])


def kernel(input, target):
    parts = _bpmll_partials(input, target.astype(jnp.int32))
    return parts.sum()
